# baseline (speedup 1.0000x reference)
import functools

import jax
import jax.numpy as jnp
from jax import lax
from jax.experimental import pallas as pl
from jax.experimental.pallas import tpu as pltpu
from jax.experimental.pallas import tpu_sc as plsc

_S = 4096
_D = 4096
_NC = 2
_NS = 16
_NW = _NC * _NS
_ROWS_PER_W = _S // _NW
_CH = 2
_NCHUNK = _ROWS_PER_W // _CH
_NSLOT = 4
_LANES = 16
_UNROLL = 8

_mesh = plsc.VectorSubcoreMesh(core_axis_name="c", subcore_axis_name="s")

_VBUF = pltpu.VMEM((_CH, _D), jnp.float32)


@functools.partial(
    pl.kernel,
    mesh=_mesh,
    out_type=jax.ShapeDtypeStruct((_S, _D), jnp.float32),
    scratch_types=(
        [_VBUF] * 12
        + [pltpu.SemaphoreType.DMA] * 8
    ),
)
def _sc_add(x_hbm, p_hbm, o_hbm, *bufs):
    wid = lax.axis_index("s") * _NC + lax.axis_index("c")
    base = wid * _ROWS_PER_W
    xv = bufs[0:4]
    pv = bufs[4:8]
    ov = bufs[8:12]
    ins = bufs[12:16]
    outs = bufs[16:20]

    def start_in(chunk, b):
        rb = base + chunk * _CH
        pltpu.async_copy(x_hbm.at[pl.ds(rb, _CH)], xv[b], ins[b])
        pltpu.async_copy(p_hbm.at[pl.ds(rb, _CH)], pv[b], ins[b])

    def wait_in(b):
        pltpu.make_async_copy(x_hbm.at[pl.ds(base, _CH)], xv[b], ins[b]).wait()
        pltpu.make_async_copy(p_hbm.at[pl.ds(base, _CH)], pv[b], ins[b]).wait()

    def start_out(chunk, b):
        rb = base + chunk * _CH
        pltpu.async_copy(ov[b], o_hbm.at[pl.ds(rb, _CH)], outs[b])

    def wait_out(b):
        pltpu.make_async_copy(
            ov[b], o_hbm.at[pl.ds(base, _CH)], outs[b]).wait()

    for b in range(_NSLOT):
        start_in(b, b)

    def group_body(g, carry):
        for b in range(_NSLOT):
            chunk = _NSLOT * g + b
            wait_in(b)

            @pl.when(chunk >= _NSLOT)
            def _():
                wait_out(b)

            for r in range(_CH):
                def vec_body(j, carry2):
                    c = j * (_LANES * _UNROLL)
                    for u in range(_UNROLL):
                        s = pl.ds(c + u * _LANES, _LANES)
                        ov[b][r, s] = xv[b][r, s] + pv[b][r, s]
                    return carry2

                lax.fori_loop(0, _D // (_LANES * _UNROLL), vec_body, 0)

            start_out(chunk, b)

            @pl.when(chunk + _NSLOT < _NCHUNK)
            def _():
                start_in(chunk + _NSLOT, b)
        return carry

    lax.fori_loop(0, _NCHUNK // _NSLOT, group_body, 0)
    for b in range(_NSLOT):
        wait_out(b)


def kernel(x, pos_table):
    return _sc_add(x, pos_table)
